# baseline (speedup 1.0000x reference)
import functools

import jax
import jax.numpy as jnp
from jax import lax
from jax.experimental import pallas as pl
from jax.experimental.pallas import tpu as pltpu
from jax.experimental.pallas import tpu_sc as plsc

NUM_CORES = 2
NUM_SUBCORES = 16
NUM_WORKERS = NUM_CORES * NUM_SUBCORES
CHUNK = 128
LANES = 16


def _make_segsum(n_rows, d, e_pad, with_counts):
  epw = e_pad // NUM_WORKERS
  n_pairs = epw // (2 * CHUNK)
  rows_per_sub = n_rows // NUM_SUBCORES

  mesh = plsc.VectorSubcoreMesh(core_axis_name="c", subcore_axis_name="s")

  out_type = [jax.ShapeDtypeStruct((NUM_CORES * n_rows, d), jnp.float32)]
  scratch = [
      pltpu.VMEM((CHUNK,), jnp.int32),
      pltpu.VMEM((CHUNK,), jnp.int32),
      pltpu.VMEM((CHUNK, d), jnp.float32),
      pltpu.VMEM_SHARED((n_rows, d), jnp.float32),
      pltpu.SemaphoreType.DMA,
  ]
  if with_counts:
    out_type.append(
        jax.ShapeDtypeStruct((NUM_WORKERS * n_rows,), jnp.float32))
    scratch.append(pltpu.VMEM((n_rows,), jnp.float32))

  @functools.partial(
      pl.kernel,
      out_type=tuple(out_type),
      mesh=mesh,
      scratch_types=scratch,
      compiler_params=pltpu.CompilerParams(needs_layout_passes=False),
  )
  def segsum(*refs):
    cnt_v = None
    if with_counts:
      (feat_hbm, src_hbm, dst_hbm, zf_hbm,
       sum_out, cnt_out,
       sidx, didx, rows, acc_sh, sem, cnt_v) = refs
    else:
      (feat_hbm, src_hbm, dst_hbm, zf_hbm,
       sum_out,
       sidx, didx, rows, acc_sh, sem) = refs

    cid = lax.axis_index("c")
    sid = lax.axis_index("s")
    wid = cid * NUM_SUBCORES + sid

    base_r = sid * rows_per_sub
    pltpu.sync_copy(zf_hbm.at[pl.ds(base_r, rows_per_sub)],
                    acc_sh.at[pl.ds(base_r, rows_per_sub)])
    if with_counts:
      zeros16 = jnp.zeros((LANES,), jnp.float32)

      def zbody(j, carry):
        cnt_v[pl.ds(j * LANES, LANES)] = zeros16
        return carry

      lax.fori_loop(0, n_rows // LANES, zbody, 0)
    plsc.subcore_barrier()

    ebase = wid * epw

    ones16 = jnp.ones((LANES,), jnp.float32)

    def body(i, carry):
      off = ebase + i * CHUNK
      pltpu.sync_copy(src_hbm.at[pl.ds(off, CHUNK)], sidx)
      pltpu.sync_copy(dst_hbm.at[pl.ds(off, CHUNK)], didx)
      pltpu.async_copy(feat_hbm.at[sidx], rows, sem).wait()
      pltpu.sync_copy(rows, acc_sh.at[didx], add=True)
      if with_counts:
        for k in range(CHUNK // LANES):
          idxv = didx[pl.ds(k * LANES, LANES)]
          plsc.addupdate_scatter(cnt_v, [idxv], ones16)
      return carry

    lax.fori_loop(0, 2 * n_pairs, body, 0)
    plsc.subcore_barrier()

    out_base = cid * n_rows + base_r
    pltpu.sync_copy(acc_sh.at[pl.ds(base_r, rows_per_sub)],
                    sum_out.at[pl.ds(out_base, rows_per_sub)])
    if with_counts:
      pltpu.sync_copy(cnt_v, cnt_out.at[pl.ds(wid * n_rows, n_rows)])

  return segsum


def _dense_body(relu, s0_ref, s1_ref, c_ref, x_ref, wlt_ref, wrt_ref,
                b_ref, o_ref):
  s = s0_ref[...] + s1_ref[...]
  c = jnp.sum(c_ref[...], axis=0)[:, None]
  mean = s / jnp.maximum(c, 1.0)
  h = (jnp.dot(mean, wlt_ref[...], preferred_element_type=jnp.float32)
       + jnp.dot(x_ref[...], wrt_ref[...], preferred_element_type=jnp.float32)
       + b_ref[...])
  o_ref[...] = jnp.maximum(h, 0.0) if relu else h


def _dense(summed, cntw, x_pad, wl, wr, b, relu, n_rows, block_rows):
  d = x_pad.shape[1]
  grid = (n_rows // block_rows,)
  off = n_rows // block_rows
  body = functools.partial(_dense_body, relu)
  return pl.pallas_call(
      body,
      grid=grid,
      in_specs=[
          pl.BlockSpec((block_rows, d), lambda i: (i, 0)),
          pl.BlockSpec((block_rows, d), lambda i: (i + off, 0)),
          pl.BlockSpec((NUM_WORKERS, block_rows), lambda i: (0, i)),
          pl.BlockSpec((block_rows, d), lambda i: (i, 0)),
          pl.BlockSpec((d, d), lambda i: (0, 0)),
          pl.BlockSpec((d, d), lambda i: (0, 0)),
          pl.BlockSpec((1, d), lambda i: (0, 0)),
      ],
      out_specs=pl.BlockSpec((block_rows, d), lambda i: (i, 0)),
      out_shape=jax.ShapeDtypeStruct((n_rows, d), jnp.float32),
  )(summed, summed, cntw, x_pad, wl.T, wr.T, b.reshape(1, d))


def kernel(x, edge_index, W1l, b1l, W1r, W2l, b2l, W2r):
  n, d = x.shape
  e = edge_index.shape[1]

  epw = -(-e // NUM_WORKERS)
  epw_pad = -(-epw // (2 * CHUNK)) * (2 * CHUNK)
  e_pad = epw_pad * NUM_WORKERS
  blk = NUM_SUBCORES * CHUNK
  n_rows = -(-(n + 1) // blk) * blk

  src = edge_index[0].astype(jnp.int32)
  dst = edge_index[1].astype(jnp.int32)
  pad = e_pad - e
  src_p = jnp.concatenate([src, jnp.zeros((pad,), jnp.int32)])
  dst_p = jnp.concatenate([dst, jnp.full((pad,), n, jnp.int32)])

  zf = jnp.zeros((n_rows, d), jnp.float32)

  segsum_cnt = _make_segsum(n_rows, d, e_pad, with_counts=True)
  segsum = _make_segsum(n_rows, d, e_pad, with_counts=False)

  summed1, cntw = segsum_cnt(x, src_p, dst_p, zf)
  cntw2 = cntw.reshape(NUM_WORKERS, n_rows)
  x_pad = jnp.concatenate([x, jnp.zeros((n_rows - n, d), x.dtype)])
  h = _dense(summed1, cntw2, x_pad, W1l, W1r, b1l, relu=True,
             n_rows=n_rows, block_rows=1024)
  (summed2,) = segsum(h, src_p, dst_p, zf)
  out = _dense(summed2, cntw2, h, W2l, W2r, b2l, relu=False,
               n_rows=n_rows, block_rows=1024)
  return out[:n]


# speedup vs baseline: 1.0056x; 1.0056x over previous
import functools

import jax
import jax.numpy as jnp
from jax import lax
from jax.experimental import pallas as pl
from jax.experimental.pallas import tpu as pltpu
from jax.experimental.pallas import tpu_sc as plsc

NUM_CORES = 2
NUM_SUBCORES = 16
NUM_WORKERS = NUM_CORES * NUM_SUBCORES
CHUNK = 128
LANES = 16


def _make_segsum(n_rows, d, e_pad, with_counts):
  epw = e_pad // NUM_WORKERS
  n_pairs = epw // (2 * CHUNK)
  rows_per_sub = n_rows // NUM_SUBCORES

  mesh = plsc.VectorSubcoreMesh(core_axis_name="c", subcore_axis_name="s")

  out_type = [jax.ShapeDtypeStruct((NUM_CORES * n_rows, d), jnp.float32)]
  scratch = [
      pltpu.VMEM((CHUNK,), jnp.int32),
      pltpu.VMEM((CHUNK,), jnp.int32),
      pltpu.VMEM((CHUNK, d), jnp.float32),
      pltpu.VMEM_SHARED((n_rows, d), jnp.float32),
      pltpu.SemaphoreType.DMA,
  ]
  if with_counts:
    out_type.append(
        jax.ShapeDtypeStruct((NUM_WORKERS * n_rows,), jnp.float32))
    scratch.append(pltpu.VMEM((n_rows,), jnp.float32))

  @functools.partial(
      pl.kernel,
      out_type=tuple(out_type),
      mesh=mesh,
      scratch_types=scratch,
      compiler_params=pltpu.CompilerParams(needs_layout_passes=False),
  )
  def segsum(*refs):
    cnt_v = None
    if with_counts:
      (feat_hbm, src_hbm, dst_hbm, zf_hbm,
       sum_out, cnt_out,
       sidx, didx, rows, acc_sh, sem, cnt_v) = refs
    else:
      (feat_hbm, src_hbm, dst_hbm, zf_hbm,
       sum_out,
       sidx, didx, rows, acc_sh, sem) = refs

    cid = lax.axis_index("c")
    sid = lax.axis_index("s")
    wid = cid * NUM_SUBCORES + sid

    base_r = sid * rows_per_sub
    pltpu.sync_copy(zf_hbm.at[pl.ds(base_r, rows_per_sub)],
                    acc_sh.at[pl.ds(base_r, rows_per_sub)])
    if with_counts:
      zeros16 = jnp.zeros((LANES,), jnp.float32)

      def zbody(j, carry):
        cnt_v[pl.ds(j * LANES, LANES)] = zeros16
        return carry

      lax.fori_loop(0, n_rows // LANES, zbody, 0)
    plsc.subcore_barrier()

    ebase = wid * epw

    ones16 = jnp.ones((LANES,), jnp.float32)

    def body(i, carry):
      off = ebase + i * CHUNK
      pltpu.sync_copy(src_hbm.at[pl.ds(off, CHUNK)], sidx)
      pltpu.sync_copy(dst_hbm.at[pl.ds(off, CHUNK)], didx)
      pltpu.async_copy(feat_hbm.at[sidx], rows, sem).wait()
      pltpu.sync_copy(rows, acc_sh.at[didx], add=True)
      if with_counts:
        for k in range(CHUNK // LANES):
          idxv = didx[pl.ds(k * LANES, LANES)]
          plsc.addupdate_scatter(cnt_v, [idxv], ones16)
      return carry

    lax.fori_loop(0, 2 * n_pairs, body, 0)
    plsc.subcore_barrier()

    out_base = cid * n_rows + base_r
    pltpu.sync_copy(acc_sh.at[pl.ds(base_r, rows_per_sub)],
                    sum_out.at[pl.ds(out_base, rows_per_sub)])
    if with_counts:
      pltpu.sync_copy(cnt_v, cnt_out.at[pl.ds(wid * n_rows, n_rows)])

  return segsum


def _dense_body(relu, s0_ref, s1_ref, c_ref, x_ref, wlt_ref, wrt_ref,
                b_ref, o_ref):
  s = s0_ref[...] + s1_ref[...]
  c = jnp.sum(c_ref[...], axis=0)[:, None]
  mean = s / jnp.maximum(c, 1.0)
  h = (jnp.dot(mean, wlt_ref[...], preferred_element_type=jnp.float32)
       + jnp.dot(x_ref[...], wrt_ref[...], preferred_element_type=jnp.float32)
       + b_ref[...])
  o_ref[...] = jnp.maximum(h, 0.0) if relu else h


def _dense(summed, cntw, x_pad, wl, wr, b, relu, n_rows, block_rows):
  d = x_pad.shape[1]
  grid = (n_rows // block_rows,)
  off = n_rows // block_rows
  body = functools.partial(_dense_body, relu)
  return pl.pallas_call(
      body,
      grid=grid,
      in_specs=[
          pl.BlockSpec((block_rows, d), lambda i: (i, 0)),
          pl.BlockSpec((block_rows, d), lambda i: (i + off, 0)),
          pl.BlockSpec((NUM_WORKERS, block_rows), lambda i: (0, i)),
          pl.BlockSpec((block_rows, d), lambda i: (i, 0)),
          pl.BlockSpec((d, d), lambda i: (0, 0)),
          pl.BlockSpec((d, d), lambda i: (0, 0)),
          pl.BlockSpec((1, d), lambda i: (0, 0)),
      ],
      out_specs=pl.BlockSpec((block_rows, d), lambda i: (i, 0)),
      out_shape=jax.ShapeDtypeStruct((n_rows, d), jnp.float32),
  )(summed, summed, cntw, x_pad, wl.T, wr.T, b.reshape(1, d))


def kernel(x, edge_index, W1l, b1l, W1r, W2l, b2l, W2r):
  n, d = x.shape
  e = edge_index.shape[1]

  epw = -(-e // NUM_WORKERS)
  epw_pad = -(-epw // (2 * CHUNK)) * (2 * CHUNK)
  e_pad = epw_pad * NUM_WORKERS
  blk = NUM_SUBCORES * CHUNK
  n_rows = -(-(n + 1) // blk) * blk

  src = edge_index[0].astype(jnp.int32)
  dst = edge_index[1].astype(jnp.int32)
  pad = e_pad - e
  src_p = jnp.concatenate([src, jnp.zeros((pad,), jnp.int32)])
  sent = n + jnp.arange(pad, dtype=jnp.int32) % jnp.int32(n_rows - n)
  dst_p = jnp.concatenate([dst, sent])

  zf = jnp.zeros((n_rows, d), jnp.float32)

  segsum_cnt = _make_segsum(n_rows, d, e_pad, with_counts=True)
  segsum = _make_segsum(n_rows, d, e_pad, with_counts=False)

  summed1, cntw = segsum_cnt(x, src_p, dst_p, zf)
  cntw2 = cntw.reshape(NUM_WORKERS, n_rows)
  x_pad = jnp.concatenate([x, jnp.zeros((n_rows - n, d), x.dtype)])
  h = _dense(summed1, cntw2, x_pad, W1l, W1r, b1l, relu=True,
             n_rows=n_rows, block_rows=1024)
  (summed2,) = segsum(h, src_p, dst_p, zf)
  out = _dense(summed2, cntw2, h, W2l, W2r, b2l, relu=False,
               n_rows=n_rows, block_rows=1024)
  return out[:n]


# speedup vs baseline: 1.5654x; 1.5566x over previous
import functools

import jax
import jax.numpy as jnp
from jax import lax
from jax.experimental import pallas as pl
from jax.experimental.pallas import tpu as pltpu
from jax.experimental.pallas import tpu_sc as plsc

NUM_CORES = 2
NUM_SUBCORES = 16
NUM_WORKERS = NUM_CORES * NUM_SUBCORES
CHUNK = 128
LANES = 16


def _make_segsum(n_rows, d, e_pad, with_counts):
  epw = e_pad // NUM_WORKERS
  n_chunks = epw // CHUNK
  rows_per_sub = n_rows // NUM_SUBCORES

  mesh = plsc.VectorSubcoreMesh(core_axis_name="c", subcore_axis_name="s")

  out_type = [jax.ShapeDtypeStruct((NUM_CORES * n_rows, d), jnp.float32)]
  scratch = [
      pltpu.VMEM((CHUNK,), jnp.int32),
      pltpu.VMEM((CHUNK,), jnp.int32),
      pltpu.VMEM((CHUNK, d), jnp.float32),
      pltpu.VMEM_SHARED((n_rows, d), jnp.float32),
      pltpu.SemaphoreType.DMA,
  ]
  if with_counts:
    out_type.append(
        jax.ShapeDtypeStruct((NUM_WORKERS * n_rows,), jnp.float32))
    scratch.append(pltpu.VMEM((n_rows,), jnp.float32))

  @functools.partial(
      pl.kernel,
      out_type=tuple(out_type),
      mesh=mesh,
      scratch_types=scratch,
      compiler_params=pltpu.CompilerParams(needs_layout_passes=False),
  )
  def segsum(*refs):
    cnt_v = None
    if with_counts:
      (feat_hbm, src_hbm, dst_hbm, zf_hbm,
       sum_out, cnt_out,
       sidx, didx, rows, acc_sh, sem, cnt_v) = refs
    else:
      (feat_hbm, src_hbm, dst_hbm, zf_hbm,
       sum_out,
       sidx, didx, rows, acc_sh, sem) = refs

    cid = lax.axis_index("c")
    sid = lax.axis_index("s")
    wid = cid * NUM_SUBCORES + sid

    base_r = sid * rows_per_sub
    pltpu.sync_copy(zf_hbm.at[pl.ds(base_r, rows_per_sub)],
                    acc_sh.at[pl.ds(base_r, rows_per_sub)])
    if with_counts:
      zeros16 = jnp.zeros((LANES,), jnp.float32)

      def zbody(j, carry):
        cnt_v[pl.ds(j * LANES, LANES)] = zeros16
        return carry

      lax.fori_loop(0, n_rows // LANES, zbody, 0)
    plsc.subcore_barrier()

    ebase = wid * epw

    ones16 = jnp.ones((LANES,), jnp.float32)

    def body(i, carry):
      off = ebase + i * CHUNK
      pltpu.sync_copy(src_hbm.at[pl.ds(off, CHUNK)], sidx)
      pltpu.sync_copy(dst_hbm.at[pl.ds(off, CHUNK)], didx)
      pltpu.async_copy(feat_hbm.at[sidx], rows, sem).wait()
      pltpu.sync_copy(rows, acc_sh.at[didx], add=True)
      if with_counts:
        for k in range(CHUNK // LANES):
          idxv = didx[pl.ds(k * LANES, LANES)]
          plsc.addupdate_scatter(cnt_v, [idxv], ones16)
      return carry

    lax.fori_loop(0, n_chunks, body, 0)
    plsc.subcore_barrier()

    out_base = cid * n_rows + base_r
    pltpu.sync_copy(acc_sh.at[pl.ds(base_r, rows_per_sub)],
                    sum_out.at[pl.ds(out_base, rows_per_sub)])
    if with_counts:
      pltpu.sync_copy(cnt_v, cnt_out.at[pl.ds(wid * n_rows, n_rows)])

  return segsum


def _dense_body(relu, s0_ref, s1_ref, c_ref, x_ref, wlt_ref, wrt_ref,
                b_ref, o_ref):
  s = s0_ref[...] + s1_ref[...]
  c = jnp.sum(c_ref[...], axis=0)[:, None]
  mean = s / jnp.maximum(c, 1.0)
  h = (jnp.dot(mean, wlt_ref[...], preferred_element_type=jnp.float32)
       + jnp.dot(x_ref[...], wrt_ref[...], preferred_element_type=jnp.float32)
       + b_ref[...])
  o_ref[...] = jnp.maximum(h, 0.0) if relu else h


def _dense(summed, cntw, x_pad, wl, wr, b, relu, n_rows, block_rows):
  d = x_pad.shape[1]
  grid = (n_rows // block_rows,)
  off = n_rows // block_rows
  body = functools.partial(_dense_body, relu)
  return pl.pallas_call(
      body,
      grid=grid,
      in_specs=[
          pl.BlockSpec((block_rows, d), lambda i: (i, 0)),
          pl.BlockSpec((block_rows, d), lambda i: (i + off, 0)),
          pl.BlockSpec((NUM_WORKERS, block_rows), lambda i: (0, i)),
          pl.BlockSpec((block_rows, d), lambda i: (i, 0)),
          pl.BlockSpec((d, d), lambda i: (0, 0)),
          pl.BlockSpec((d, d), lambda i: (0, 0)),
          pl.BlockSpec((1, d), lambda i: (0, 0)),
      ],
      out_specs=pl.BlockSpec((block_rows, d), lambda i: (i, 0)),
      out_shape=jax.ShapeDtypeStruct((n_rows, d), jnp.float32),
  )(summed, summed, cntw, x_pad, wl.T, wr.T, b.reshape(1, d))


def kernel(x, edge_index, W1l, b1l, W1r, W2l, b2l, W2r):
  n, d = x.shape
  e = edge_index.shape[1]

  epw = -(-e // NUM_WORKERS)
  epw_pad = -(-epw // CHUNK) * CHUNK
  e_pad = epw_pad * NUM_WORKERS
  blk = NUM_SUBCORES * CHUNK
  n_rows = -(-(n + 1) // blk) * blk

  src = edge_index[0].astype(jnp.int32)
  dst = edge_index[1].astype(jnp.int32)
  pad = e_pad - e
  src_p = jnp.concatenate([src, jnp.zeros((pad,), jnp.int32)])
  sent = n + jnp.arange(pad, dtype=jnp.int32) % jnp.int32(n_rows - n)
  dst_p = jnp.concatenate([dst, sent])

  zf = jnp.zeros((n_rows, d), jnp.float32)

  segsum_cnt = _make_segsum(n_rows, d, e_pad, with_counts=True)
  segsum = _make_segsum(n_rows, d, e_pad, with_counts=False)

  summed1, cntw = segsum_cnt(x, src_p, dst_p, zf)
  cntw2 = cntw.reshape(NUM_WORKERS, n_rows)
  x_pad = jnp.concatenate([x, jnp.zeros((n_rows - n, d), x.dtype)])
  h = _dense(summed1, cntw2, x_pad, W1l, W1r, b1l, relu=True,
             n_rows=n_rows, block_rows=1024)
  (summed2,) = segsum(h, src_p, dst_p, zf)
  out = _dense(summed2, cntw2, h, W2l, W2r, b2l, relu=False,
               n_rows=n_rows, block_rows=1024)
  return out[:n]


# speedup vs baseline: 2.0439x; 1.3057x over previous
import functools

import jax
import jax.numpy as jnp
from jax import lax
from jax.experimental import pallas as pl
from jax.experimental.pallas import tpu as pltpu
from jax.experimental.pallas import tpu_sc as plsc

NUM_CORES = 2
NUM_SUBCORES = 16
NUM_WORKERS = NUM_CORES * NUM_SUBCORES
CHUNK = 128
LANES = 16


def _make_segsum(n_rows, d, e_pad, with_counts):
  epw = e_pad // NUM_WORKERS
  n_chunks = epw // CHUNK
  rows_per_sub = n_rows // NUM_SUBCORES

  mesh = plsc.VectorSubcoreMesh(core_axis_name="c", subcore_axis_name="s")

  out_type = [jax.ShapeDtypeStruct((NUM_CORES * n_rows, d), jnp.float32)]
  scratch = [
      pltpu.VMEM((CHUNK,), jnp.int32),
      pltpu.VMEM((CHUNK,), jnp.int32),
      pltpu.VMEM((CHUNK, d), jnp.float32),
      pltpu.SemaphoreType.DMA,
      pltpu.VMEM((CHUNK,), jnp.int32),
      pltpu.VMEM((CHUNK,), jnp.int32),
      pltpu.VMEM((CHUNK, d), jnp.float32),
      pltpu.SemaphoreType.DMA,
      pltpu.VMEM_SHARED((n_rows, d), jnp.float32),
  ]
  if with_counts:
    out_type.append(
        jax.ShapeDtypeStruct((NUM_WORKERS * n_rows,), jnp.float32))
    scratch.append(pltpu.VMEM((n_rows,), jnp.float32))

  @functools.partial(
      pl.kernel,
      out_type=tuple(out_type),
      mesh=mesh,
      scratch_types=scratch,
      compiler_params=pltpu.CompilerParams(needs_layout_passes=False),
  )
  def segsum(*refs):
    cnt_v = None
    if with_counts:
      (feat_hbm, src_hbm, dst_hbm, zf_hbm,
       sum_out, cnt_out,
       sidx_a, didx_a, rows_a, sem_a,
       sidx_b, didx_b, rows_b, sem_b, acc_sh, cnt_v) = refs
    else:
      (feat_hbm, src_hbm, dst_hbm, zf_hbm,
       sum_out,
       sidx_a, didx_a, rows_a, sem_a,
       sidx_b, didx_b, rows_b, sem_b, acc_sh) = refs

    cid = lax.axis_index("c")
    sid = lax.axis_index("s")
    wid = cid * NUM_SUBCORES + sid

    base_r = sid * rows_per_sub
    pltpu.sync_copy(zf_hbm.at[pl.ds(base_r, rows_per_sub)],
                    acc_sh.at[pl.ds(base_r, rows_per_sub)])
    if with_counts:
      zeros16 = jnp.zeros((LANES,), jnp.float32)

      def zbody(j, carry):
        cnt_v[pl.ds(j * LANES, LANES)] = zeros16
        return carry

      lax.fori_loop(0, n_rows // LANES, zbody, 0)
    plsc.subcore_barrier()

    ebase = wid * epw

    ones16 = jnp.ones((LANES,), jnp.float32)

    def load_and_fire(off, sidx, didx, rows, sem):
      pltpu.sync_copy(src_hbm.at[pl.ds(off, CHUNK)], sidx)
      pltpu.sync_copy(dst_hbm.at[pl.ds(off, CHUNK)], didx)
      pltpu.async_copy(feat_hbm.at[sidx], rows, sem)

    def drain_and_scatter(sidx, didx, rows, sem):
      pltpu.make_async_copy(feat_hbm.at[sidx], rows, sem).wait()
      pltpu.sync_copy(rows, acc_sh.at[didx], add=True)
      if with_counts:
        for k in range(CHUNK // LANES):
          idxv = didx[pl.ds(k * LANES, LANES)]
          plsc.addupdate_scatter(cnt_v, [idxv], ones16)

    nloop = (n_chunks - 1) // 2

    load_and_fire(ebase, sidx_a, didx_a, rows_a, sem_a)

    def body(j, carry):
      off = ebase + j * 2 * CHUNK
      load_and_fire(off + CHUNK, sidx_b, didx_b, rows_b, sem_b)
      drain_and_scatter(sidx_a, didx_a, rows_a, sem_a)
      load_and_fire(off + 2 * CHUNK, sidx_a, didx_a, rows_a, sem_a)
      drain_and_scatter(sidx_b, didx_b, rows_b, sem_b)
      return carry

    lax.fori_loop(0, nloop, body, 0)
    drain_and_scatter(sidx_a, didx_a, rows_a, sem_a)
    if n_chunks % 2 == 0:
      off = ebase + (n_chunks - 1) * CHUNK
      load_and_fire(off, sidx_b, didx_b, rows_b, sem_b)
      drain_and_scatter(sidx_b, didx_b, rows_b, sem_b)
    plsc.subcore_barrier()

    out_base = cid * n_rows + base_r
    pltpu.sync_copy(acc_sh.at[pl.ds(base_r, rows_per_sub)],
                    sum_out.at[pl.ds(out_base, rows_per_sub)])
    if with_counts:
      pltpu.sync_copy(cnt_v, cnt_out.at[pl.ds(wid * n_rows, n_rows)])

  return segsum


def _dense_body(relu, s0_ref, s1_ref, c_ref, x_ref, wlt_ref, wrt_ref,
                b_ref, o_ref):
  s = s0_ref[...] + s1_ref[...]
  c = jnp.sum(c_ref[...], axis=0)[:, None]
  mean = s / jnp.maximum(c, 1.0)
  h = (jnp.dot(mean, wlt_ref[...], preferred_element_type=jnp.float32)
       + jnp.dot(x_ref[...], wrt_ref[...], preferred_element_type=jnp.float32)
       + b_ref[...])
  o_ref[...] = jnp.maximum(h, 0.0) if relu else h


def _dense(summed, cntw, x_pad, wl, wr, b, relu, n_rows, block_rows):
  d = x_pad.shape[1]
  grid = (n_rows // block_rows,)
  off = n_rows // block_rows
  body = functools.partial(_dense_body, relu)
  return pl.pallas_call(
      body,
      grid=grid,
      in_specs=[
          pl.BlockSpec((block_rows, d), lambda i: (i, 0)),
          pl.BlockSpec((block_rows, d), lambda i: (i + off, 0)),
          pl.BlockSpec((NUM_WORKERS, block_rows), lambda i: (0, i)),
          pl.BlockSpec((block_rows, d), lambda i: (i, 0)),
          pl.BlockSpec((d, d), lambda i: (0, 0)),
          pl.BlockSpec((d, d), lambda i: (0, 0)),
          pl.BlockSpec((1, d), lambda i: (0, 0)),
      ],
      out_specs=pl.BlockSpec((block_rows, d), lambda i: (i, 0)),
      out_shape=jax.ShapeDtypeStruct((n_rows, d), jnp.float32),
  )(summed, summed, cntw, x_pad, wl.T, wr.T, b.reshape(1, d))


def kernel(x, edge_index, W1l, b1l, W1r, W2l, b2l, W2r):
  n, d = x.shape
  e = edge_index.shape[1]

  epw = -(-e // NUM_WORKERS)
  epw_pad = -(-epw // CHUNK) * CHUNK
  e_pad = epw_pad * NUM_WORKERS
  blk = NUM_SUBCORES * CHUNK
  n_rows = -(-(n + 1) // blk) * blk

  src = edge_index[0].astype(jnp.int32)
  dst = edge_index[1].astype(jnp.int32)
  pad = e_pad - e
  src_p = jnp.concatenate([src, jnp.zeros((pad,), jnp.int32)])
  sent = n + jnp.arange(pad, dtype=jnp.int32) % jnp.int32(n_rows - n)
  dst_p = jnp.concatenate([dst, sent])

  zf = jnp.zeros((n_rows, d), jnp.float32)

  segsum_cnt = _make_segsum(n_rows, d, e_pad, with_counts=True)
  segsum = _make_segsum(n_rows, d, e_pad, with_counts=False)

  summed1, cntw = segsum_cnt(x, src_p, dst_p, zf)
  cntw2 = cntw.reshape(NUM_WORKERS, n_rows)
  x_pad = jnp.concatenate([x, jnp.zeros((n_rows - n, d), x.dtype)])
  h = _dense(summed1, cntw2, x_pad, W1l, W1r, b1l, relu=True,
             n_rows=n_rows, block_rows=1024)
  (summed2,) = segsum(h, src_p, dst_p, zf)
  out = _dense(summed2, cntw2, h, W2l, W2r, b2l, relu=False,
               n_rows=n_rows, block_rows=1024)
  return out[:n]
